# Initial kernel scaffold; baseline (speedup 1.0000x reference)
import jax
import jax.numpy as jnp
from jax.experimental import pallas as pl


def kernel(weight, words):
    raise NotImplementedError("write your pallas kernel here")


# speedup vs baseline: 1.1194x; 1.1194x over previous
import functools

import jax
import jax.numpy as jnp
from jax import lax
from jax.experimental import pallas as pl
from jax.experimental.pallas import tpu as pltpu
from jax.experimental.pallas import tpu_sc as plsc

DROPOUT = 0.1
NC = 2
NS = 16
NW = NC * NS
LANES = 16


def _sc_gather_scale(weight, scale, idx, chunk):
    n = idx.shape[0]
    d = weight.shape[1]
    b_per_w = n // NW
    n_chunks = b_per_w // chunk
    mesh = plsc.VectorSubcoreMesh(core_axis_name="c", subcore_axis_name="s")

    @functools.partial(
        pl.kernel,
        out_type=jax.ShapeDtypeStruct((n, d), jnp.float32),
        mesh=mesh,
        scratch_types=[
            pltpu.VMEM((chunk,), jnp.int32),
            pltpu.VMEM((chunk,), jnp.float32),
            pltpu.VMEM((chunk, d), jnp.float32),
            pltpu.SemaphoreType.DMA,
            pltpu.SemaphoreType.DMA,
        ],
        compiler_params=pltpu.CompilerParams(use_tc_tiling_on_sc=False),
    )
    def run(weight_hbm, scale_hbm, idx_hbm, out_hbm, idx_v, sc_v, rows_v,
            sem_w, sem_s):
        wid = lax.axis_index("s") * NC + lax.axis_index("c")
        base = wid * b_per_w

        def chunk_body(k, carry):
            off = base + k * chunk
            pltpu.sync_copy(idx_hbm.at[pl.ds(off, chunk)], idx_v)
            cp_w = pltpu.async_copy(weight_hbm.at[idx_v], rows_v, sem_w)
            cp_s = pltpu.async_copy(scale_hbm.at[idx_v], sc_v, sem_s)
            cp_w.wait()
            cp_s.wait()

            def grp_body(g, c2):
                r0 = g * LANES
                svec = sc_v[pl.ds(r0, LANES)]
                for j in range(LANES):
                    sv = jnp.full((LANES,), svec[j], jnp.float32)
                    for col in range(d // LANES):
                        sl = pl.ds(col * LANES, LANES)
                        rows_v[r0 + j, sl] = rows_v[r0 + j, sl] * sv
                return c2

            lax.fori_loop(0, chunk // LANES, grp_body, 0)
            pltpu.sync_copy(rows_v, out_hbm.at[pl.ds(off, chunk)])
            return carry

        lax.fori_loop(0, n_chunks, chunk_body, 0)

    return run(weight, scale, idx)


def kernel(weight, words):
    vocab = weight.shape[0]
    d = weight.shape[1]
    mask_key = jax.random.fold_in(jax.random.key(0), 1)
    keep = jax.random.bernoulli(
        mask_key, 1.0 - DROPOUT, (vocab, 1)).astype(weight.dtype)
    scale = (keep / (1.0 - DROPOUT)).reshape(vocab)

    idx = words.reshape(-1).astype(jnp.int32)
    out = _sc_gather_scale(weight, scale, idx, chunk=1600)
    return out.reshape(words.shape + (d,))


# speedup vs baseline: 1.4631x; 1.3071x over previous
import functools

import jax
import jax.numpy as jnp
from jax import lax
from jax.experimental import pallas as pl
from jax.experimental.pallas import tpu as pltpu
from jax.experimental.pallas import tpu_sc as plsc

DROPOUT = 0.1
NC = 2
NS = 16
NW = NC * NS
LANES = 16

VOCAB = 1000000
D = 32
B = 16384
H = 50
BPW = B // NW
NTILE_E = D // 8
SLAB = NTILE_E * (B // 128) * 1024


def _sc_embed_dropout(weight, scale, idx_t):
    mesh = plsc.VectorSubcoreMesh(core_axis_name="c", subcore_axis_name="s")

    @functools.partial(
        pl.kernel,
        out_type=jax.ShapeDtypeStruct((H * SLAB,), jnp.float32),
        mesh=mesh,
        scratch_types=[
            pltpu.VMEM((BPW,), jnp.int32),
            pltpu.VMEM((BPW,), jnp.float32),
            pltpu.VMEM((BPW, D), jnp.float32),
            pltpu.VMEM((BPW * D,), jnp.float32),
            pltpu.SemaphoreType.DMA,
            pltpu.SemaphoreType.DMA,
        ],
        compiler_params=pltpu.CompilerParams(
            use_tc_tiling_on_sc=False, needs_layout_passes=False),
    )
    def run(tbl, scale_hbm, idx_hbm, out_hbm, idx_v, sc_v, rows_v, obuf,
            sem_w, sem_s):
        wid = lax.axis_index("s") * NC + lax.axis_index("c")
        b0 = wid * BPW
        cb0 = b0 // 128
        ncb = BPW // 128

        def h_body(h, carry):
            pltpu.sync_copy(idx_hbm.at[pl.ds(h * B + b0, BPW)], idx_v)
            cp_w = pltpu.async_copy(tbl.at[idx_v], rows_v, sem_w)
            cp_s = pltpu.async_copy(scale_hbm.at[idx_v], sc_v, sem_s)
            cp_w.wait()
            cp_s.wait()

            def g_body(g, c2):
                bl = g * LANES
                svec = sc_v[pl.ds(bl, LANES)]
                row_idx = bl + jax.lax.iota(jnp.int32, LANES)
                cb = bl // 128
                bm = bl % 128
                for e in range(D):
                    col_idx = jnp.full((LANES,), e, jnp.int32)
                    val = plsc.load_gather(rows_v, [row_idx, col_idx])
                    dst = (e // 8) * (ncb * 1024) + cb * 1024 + (e % 8) * 128 + bm
                    obuf[pl.ds(dst, LANES)] = val * svec
                return c2

            lax.fori_loop(0, BPW // LANES, g_body, 0)

            base = h * SLAB + cb0 * 1024
            for re in range(NTILE_E):
                pltpu.sync_copy(
                    obuf.at[pl.ds(re * (ncb * 1024), ncb * 1024)],
                    out_hbm.at[pl.ds(base + re * (B // 128) * 1024,
                                     ncb * 1024)])
            return carry

        lax.fori_loop(0, H, h_body, 0)

    return run(weight, scale, idx_t)


def kernel(weight, words):
    mask_key = jax.random.fold_in(jax.random.key(0), 1)
    keep = jax.random.bernoulli(
        mask_key, 1.0 - DROPOUT, (VOCAB, 1)).astype(weight.dtype)
    scale = (keep / (1.0 - DROPOUT)).reshape(VOCAB)

    idx_t = words.T.reshape(-1).astype(jnp.int32)
    out_flat = _sc_embed_dropout(weight, scale, idx_t)
    t = out_flat.reshape(H, NTILE_E, B // 128, 8, 128)
    out = t.transpose(2, 4, 0, 1, 3).reshape(B, H, D)
    return out


# speedup vs baseline: 1.8399x; 1.2575x over previous
import functools

import jax
import jax.numpy as jnp
from jax import lax
from jax.experimental import pallas as pl
from jax.experimental.pallas import tpu as pltpu
from jax.experimental.pallas import tpu_sc as plsc

DROPOUT = 0.1
NC = 2
NS = 16
NW = NC * NS
LANES = 16

VOCAB = 1000000
D = 32
B = 16384
H = 50
BPW = B // NW
NTILE_E = D // 8
SLAB = NTILE_E * (B // 128) * 1024


def _sc_embed_dropout(weight, scale, idx_t):
    mesh = plsc.VectorSubcoreMesh(core_axis_name="c", subcore_axis_name="s")

    @functools.partial(
        pl.kernel,
        out_type=jax.ShapeDtypeStruct((H * SLAB,), jnp.float32),
        mesh=mesh,
        scratch_types=[
            pltpu.VMEM((BPW,), jnp.int32),
            pltpu.VMEM((BPW,), jnp.float32),
            pltpu.VMEM((BPW, D), jnp.float32),
            pltpu.VMEM((BPW * D,), jnp.float32),
            pltpu.SemaphoreType.DMA,
            pltpu.SemaphoreType.DMA,
        ],
        compiler_params=pltpu.CompilerParams(
            use_tc_tiling_on_sc=False, needs_layout_passes=False),
    )
    def run(tbl, scale_hbm, idx_hbm, out_hbm, idx_v, sc_v, rows_v, obuf,
            sem_w, sem_s):
        wid = lax.axis_index("s") * NC + lax.axis_index("c")
        b0 = wid * BPW
        cb0 = b0 // 128
        ncb = BPW // 128

        def h_body(h, carry):
            pltpu.sync_copy(idx_hbm.at[pl.ds(h * B + b0, BPW)], idx_v)
            cp_w = pltpu.async_copy(tbl.at[idx_v], rows_v, sem_w)
            cp_s = pltpu.async_copy(scale_hbm.at[idx_v], sc_v, sem_s)
            cp_w.wait()
            cp_s.wait()

            @plsc.parallel_loop(0, BPW // LANES, 1, unroll=2)
            def g_body(g):
                bl = g * LANES
                svec = sc_v[pl.ds(bl, LANES)]
                row_idx = bl + jax.lax.iota(jnp.int32, LANES)
                cb = bl // 128
                bm = bl % 128
                for e in range(D):
                    col_idx = jnp.full((LANES,), e, jnp.int32)
                    val = plsc.load_gather(rows_v, [row_idx, col_idx])
                    dst = (e // 8) * (ncb * 1024) + cb * 1024 + (e % 8) * 128 + bm
                    obuf[pl.ds(dst, LANES)] = val * svec

            base = h * SLAB + cb0 * 1024
            for re in range(NTILE_E):
                pltpu.sync_copy(
                    obuf.at[pl.ds(re * (ncb * 1024), ncb * 1024)],
                    out_hbm.at[pl.ds(base + re * (B // 128) * 1024,
                                     ncb * 1024)])
            return carry

        lax.fori_loop(0, H, h_body, 0)

    return run(weight, scale, idx_t)


def kernel(weight, words):
    mask_key = jax.random.fold_in(jax.random.key(0), 1)
    keep = jax.random.bernoulli(
        mask_key, 1.0 - DROPOUT, (VOCAB, 1)).astype(weight.dtype)
    scale = (keep / (1.0 - DROPOUT)).reshape(VOCAB)

    idx_t = words.T.reshape(-1).astype(jnp.int32)
    out_flat = _sc_embed_dropout(weight, scale, idx_t)
    t = out_flat.reshape(H, NTILE_E, B // 128, 8, 128)
    out = t.transpose(2, 4, 0, 1, 3).reshape(B, H, D)
    return out
